# baseline (speedup 1.0000x reference)
import jax
import jax.numpy as jnp
from jax import lax
from jax.experimental import pallas as pl
from jax.experimental.pallas import tpu as pltpu
from jax.experimental.pallas import tpu_sc as plsc

VOCAB = 100000
SEQ_LEN = 200
HIDDEN = 64
BATCH = 4096

NC = 2
NS = 16
NW = NC * NS
BBLK = BATCH // NW
N_BODIES = SEQ_LEN // 2


def _body(x_hbm, tok_hbm, pos_hbm, out_hbm,
          xb_v, xt_v, pe_v, in_a, in_b, img_a, img_b,
          gsem_a, gsem_b, osem_a, osem_b):
    wid = lax.axis_index("s") * NC + lax.axis_index("c")
    b0 = wid * BBLK

    pltpu.sync_copy(pos_hbm, pe_v)
    pltpu.sync_copy(x_hbm.at[pl.ds(b0, BBLK)], xb_v)

    iota = lax.iota(jnp.int32, 16)
    rowv = [iota + (16 * jj) for jj in range(BBLK // 16)]
    rotp = [(iota + d) & 15 for d in range(16)]

    @plsc.parallel_loop(0, SEQ_LEN)
    def xt_row(s):
        colv = jnp.broadcast_to(s, (16,)).astype(jnp.int32)
        for jj in range(BBLK // 16):
            v = plsc.load_gather(xb_v, [rowv[jj], colv])
            xt_v[s, pl.ds(jj * 16, 16)] = v

    def issue(s, in_v, gsem):
        pltpu.async_copy(tok_hbm.at[xt_v.at[s]], in_v, gsem)

    def out_copies(s, img_v, osem):
        return [
            pltpu.make_async_copy(
                img_v.at[pl.ds(hg * 8, 8)], out_hbm.at[s, hg, wid], osem)
            for hg in range(HIDDEN // 8)
        ]

    def process(s, in_v, img_v, gsem, osem):
        @pl.when(s >= 2)
        def _():
            for c in out_copies(s, img_v, osem):
                c.wait()

        pltpu.make_async_copy(tok_hbm.at[xt_v.at[s]], in_v, gsem).wait()

        srow = jnp.broadcast_to(s, (16,)).astype(jnp.int32)
        for hb in range(HIDDEN // 16):
            cols = [rotp[d] + (hb * 16) for d in range(16)]
            pe_rot = [
                plsc.load_gather(pe_v, [srow, cols[d]]) for d in range(16)
            ]

            @plsc.parallel_loop(0, BBLK // 16, unroll=2)
            def bb_body(bb, cols=cols, pe_rot=pe_rot):
                rows = iota + bb * 16
                for d in range(16):
                    val = plsc.load_gather(in_v, [rows, cols[d]])
                    val = val + pe_rot[d]
                    plsc.store_scatter(img_v, [cols[d], rows], val)
        for hg in range(HIDDEN // 8):
            pltpu.async_copy(
                img_v.at[pl.ds(hg * 8, 8)], out_hbm.at[s, hg, wid], osem)

    issue(jnp.int32(0), in_a, gsem_a)

    def loop_body(t, _):
        s0 = 2 * t
        s1 = s0 + 1
        issue(s1, in_b, gsem_b)
        process(s0, in_a, img_a, gsem_a, osem_a)

        @pl.when(t + 1 < N_BODIES)
        def _():
            issue(s0 + 2, in_a, gsem_a)

        process(s1, in_b, img_b, gsem_b, osem_b)
        return 0

    lax.fori_loop(0, N_BODIES, loop_body, 0)

    for c in out_copies(jnp.int32(0), img_a, osem_a):
        c.wait()
    for c in out_copies(jnp.int32(0), img_b, osem_b):
        c.wait()


@jax.jit
def _encode(x, token_table, pos_table):
    mesh = plsc.VectorSubcoreMesh(core_axis_name="c", subcore_axis_name="s")
    return pl.kernel(
        _body,
        out_type=jax.ShapeDtypeStruct(
            (SEQ_LEN, HIDDEN // 8, NW, 8, 128), jnp.float32),
        mesh=mesh,
        scratch_types=[
            pltpu.VMEM((BBLK, SEQ_LEN), jnp.int32),
            pltpu.VMEM((SEQ_LEN, BBLK), jnp.int32),
            pltpu.VMEM((SEQ_LEN, HIDDEN), jnp.float32),
            pltpu.VMEM((BBLK, HIDDEN), jnp.float32),
            pltpu.VMEM((BBLK, HIDDEN), jnp.float32),
            pltpu.VMEM((HIDDEN, 128), jnp.float32),
            pltpu.VMEM((HIDDEN, 128), jnp.float32),
            pltpu.SemaphoreType.DMA,
            pltpu.SemaphoreType.DMA,
            pltpu.SemaphoreType.DMA,
            pltpu.SemaphoreType.DMA,
        ],
        compiler_params=pltpu.CompilerParams(
            use_tc_tiling_on_sc=False, needs_layout_passes=False),
    )(x, token_table, pos_table)


def kernel(x, token_table, pos_table):
    out5 = _encode(x.astype(jnp.int32), token_table, pos_table)
    return out5.transpose(2, 4, 0, 1, 3).reshape(BATCH, SEQ_LEN, HIDDEN)


# speedup vs baseline: 2.6795x; 2.6795x over previous
import jax
import jax.numpy as jnp
from jax import lax
from jax.experimental import pallas as pl
from jax.experimental.pallas import tpu as pltpu
from jax.experimental.pallas import tpu_sc as plsc

VOCAB = 100000
SEQ_LEN = 200
HIDDEN = 64
BATCH = 4096

NC = 2
NS = 16
NW = NC * NS
BBLK = BATCH // NW
N_BODIES = SEQ_LEN // 2


def _body(x_hbm, tok_hbm, pos_hbm, out_hbm,
          xb_v, xt_v, pe_v, in_a, in_b, img_a, img_b,
          gsem_a, gsem_b, osem_a, osem_b):
    wid = lax.axis_index("s") * NC + lax.axis_index("c")
    b0 = wid * BBLK

    pltpu.sync_copy(pos_hbm, pe_v)
    pltpu.sync_copy(x_hbm.at[pl.ds(b0, BBLK)], xb_v)

    iota = lax.iota(jnp.int32, 16)
    rowv = [iota + (16 * jj) for jj in range(BBLK // 16)]
    rotp = [(iota + d) & 15 for d in range(16)]

    @plsc.parallel_loop(0, SEQ_LEN)
    def xt_row(s):
        colv = jnp.broadcast_to(s, (16,)).astype(jnp.int32)
        for jj in range(BBLK // 16):
            v = plsc.load_gather(xb_v, [rowv[jj], colv])
            xt_v[s, pl.ds(jj * 16, 16)] = v

    def issue(s, in_v, gsem):
        pltpu.async_copy(tok_hbm.at[xt_v.at[s]], in_v, gsem)

    def out_copies(s, img_v, osem):
        return [
            pltpu.make_async_copy(
                img_v.at[pl.ds(hg * 8, 8)], out_hbm.at[s, hg, wid], osem)
            for hg in range(HIDDEN // 8)
        ]

    def process(s, in_v, img_v, gsem, osem):
        @pl.when(s >= 2)
        def _():
            for c in out_copies(s, img_v, osem):
                c.wait()

        pltpu.make_async_copy(tok_hbm.at[xt_v.at[s]], in_v, gsem).wait()

        srow = jnp.broadcast_to(s, (16,)).astype(jnp.int32)
        for hb in range(HIDDEN // 16):
            cols = [rotp[d] + (hb * 16) for d in range(16)]
            pe_rot = [
                plsc.load_gather(pe_v, [srow, cols[d]]) for d in range(16)
            ]

            def bb_body(bb, _, cols=cols, pe_rot=pe_rot):
                rows = iota + bb * 16
                vals = [plsc.load_gather(in_v, [rows, cols[d]])
                        for d in range(16)]
                vals = [vals[d] + pe_rot[d] for d in range(16)]
                for d in range(16):
                    plsc.store_scatter(img_v, [cols[d], rows], vals[d])
                return 0

            lax.fori_loop(0, BBLK // 16, bb_body, 0)
        for hg in range(HIDDEN // 8):
            pltpu.async_copy(
                img_v.at[pl.ds(hg * 8, 8)], out_hbm.at[s, hg, wid], osem)

    issue(jnp.int32(0), in_a, gsem_a)

    def loop_body(t, _):
        s0 = 2 * t
        s1 = s0 + 1
        issue(s1, in_b, gsem_b)
        process(s0, in_a, img_a, gsem_a, osem_a)

        @pl.when(t + 1 < N_BODIES)
        def _():
            issue(s0 + 2, in_a, gsem_a)

        process(s1, in_b, img_b, gsem_b, osem_b)
        return 0

    lax.fori_loop(0, N_BODIES, loop_body, 0)

    for c in out_copies(jnp.int32(0), img_a, osem_a):
        c.wait()
    for c in out_copies(jnp.int32(0), img_b, osem_b):
        c.wait()


@jax.jit
def _encode(x, token_table, pos_table):
    mesh = plsc.VectorSubcoreMesh(core_axis_name="c", subcore_axis_name="s")
    return pl.kernel(
        _body,
        out_type=jax.ShapeDtypeStruct(
            (SEQ_LEN, HIDDEN // 8, NW, 8, 128), jnp.float32),
        mesh=mesh,
        scratch_types=[
            pltpu.VMEM((BBLK, SEQ_LEN), jnp.int32),
            pltpu.VMEM((SEQ_LEN, BBLK), jnp.int32),
            pltpu.VMEM((SEQ_LEN, HIDDEN), jnp.float32),
            pltpu.VMEM((BBLK, HIDDEN), jnp.float32),
            pltpu.VMEM((BBLK, HIDDEN), jnp.float32),
            pltpu.VMEM((HIDDEN, 128), jnp.float32),
            pltpu.VMEM((HIDDEN, 128), jnp.float32),
            pltpu.SemaphoreType.DMA,
            pltpu.SemaphoreType.DMA,
            pltpu.SemaphoreType.DMA,
            pltpu.SemaphoreType.DMA,
        ],
        compiler_params=pltpu.CompilerParams(
            use_tc_tiling_on_sc=False, needs_layout_passes=False),
    )(x, token_table, pos_table)


def kernel(x, token_table, pos_table):
    out5 = _encode(x.astype(jnp.int32), token_table, pos_table)
    return out5.transpose(2, 4, 0, 1, 3).reshape(BATCH, SEQ_LEN, HIDDEN)


# speedup vs baseline: 2.6820x; 1.0009x over previous
import jax
import jax.numpy as jnp
from jax import lax
from jax.experimental import pallas as pl
from jax.experimental.pallas import tpu as pltpu
from jax.experimental.pallas import tpu_sc as plsc

VOCAB = 100000
SEQ_LEN = 200
HIDDEN = 64
BATCH = 4096

NC = 2
NS = 16
NW = NC * NS
BBLK = BATCH // NW
N_BODIES = SEQ_LEN // 2


def _body(x_hbm, tok_hbm, pos_hbm, out_hbm,
          xb_v, xt_v, pe_v, in_a, in_b, img_a, img_b,
          gsem_a, gsem_b, osem_a, osem_b):
    wid = lax.axis_index("s") * NC + lax.axis_index("c")
    b0 = wid * BBLK

    pltpu.sync_copy(pos_hbm, pe_v)
    pltpu.sync_copy(x_hbm.at[pl.ds(b0, BBLK)], xb_v)

    iota = lax.iota(jnp.int32, 16)
    rowv = [iota + (16 * jj) for jj in range(BBLK // 16)]
    rotp = [(iota + d) & 15 for d in range(16)]

    def xt_blk(sb, _):
        s0 = sb * 16
        scols = [rotp[d] + s0 for d in range(16)]
        for jj in range(BBLK // 16):
            vals = [plsc.load_gather(xb_v, [rowv[jj], scols[d]])
                    for d in range(16)]
            for d in range(16):
                plsc.store_scatter(
                    xt_v, [scols[d], rowv[jj]], vals[d])
        return 0

    lax.fori_loop(0, SEQ_LEN // 16, xt_blk, 0)
    tail0 = (SEQ_LEN // 16) * 16
    for st in range(tail0, SEQ_LEN):
        colv = jnp.broadcast_to(st, (16,)).astype(jnp.int32)
        for jj in range(BBLK // 16):
            v = plsc.load_gather(xb_v, [rowv[jj], colv])
            xt_v[st, pl.ds(jj * 16, 16)] = v

    def issue(s, in_v, gsem):
        pltpu.async_copy(tok_hbm.at[xt_v.at[s]], in_v, gsem)

    def out_copies(s, img_v, osem):
        return [
            pltpu.make_async_copy(
                img_v.at[pl.ds(hg * 8, 8)], out_hbm.at[s, hg, wid], osem)
            for hg in range(HIDDEN // 8)
        ]

    def process(s, in_v, img_v, gsem, osem):
        @pl.when(s >= 2)
        def _():
            for c in out_copies(s, img_v, osem):
                c.wait()

        pltpu.make_async_copy(tok_hbm.at[xt_v.at[s]], in_v, gsem).wait()

        srow = jnp.broadcast_to(s, (16,)).astype(jnp.int32)
        for hb in range(HIDDEN // 16):
            cols = [rotp[d] + (hb * 16) for d in range(16)]
            pe_rot = [
                plsc.load_gather(pe_v, [srow, cols[d]]) for d in range(16)
            ]

            def bb_body(bb, _, cols=cols, pe_rot=pe_rot):
                rows = iota + bb * 16
                vals = [plsc.load_gather(in_v, [rows, cols[d]])
                        for d in range(16)]
                vals = [vals[d] + pe_rot[d] for d in range(16)]
                for d in range(16):
                    plsc.store_scatter(img_v, [cols[d], rows], vals[d])
                return 0

            lax.fori_loop(0, BBLK // 16, bb_body, 0)
        for hg in range(HIDDEN // 8):
            pltpu.async_copy(
                img_v.at[pl.ds(hg * 8, 8)], out_hbm.at[s, hg, wid], osem)

    issue(jnp.int32(0), in_a, gsem_a)

    def loop_body(t, _):
        s0 = 2 * t
        s1 = s0 + 1
        issue(s1, in_b, gsem_b)
        process(s0, in_a, img_a, gsem_a, osem_a)

        @pl.when(t + 1 < N_BODIES)
        def _():
            issue(s0 + 2, in_a, gsem_a)

        process(s1, in_b, img_b, gsem_b, osem_b)
        return 0

    lax.fori_loop(0, N_BODIES, loop_body, 0)

    for c in out_copies(jnp.int32(0), img_a, osem_a):
        c.wait()
    for c in out_copies(jnp.int32(0), img_b, osem_b):
        c.wait()


@jax.jit
def _encode(x, token_table, pos_table):
    mesh = plsc.VectorSubcoreMesh(core_axis_name="c", subcore_axis_name="s")
    return pl.kernel(
        _body,
        out_type=jax.ShapeDtypeStruct(
            (SEQ_LEN, HIDDEN // 8, NW, 8, 128), jnp.float32),
        mesh=mesh,
        scratch_types=[
            pltpu.VMEM((BBLK, SEQ_LEN), jnp.int32),
            pltpu.VMEM((SEQ_LEN, BBLK), jnp.int32),
            pltpu.VMEM((SEQ_LEN, HIDDEN), jnp.float32),
            pltpu.VMEM((BBLK, HIDDEN), jnp.float32),
            pltpu.VMEM((BBLK, HIDDEN), jnp.float32),
            pltpu.VMEM((HIDDEN, 128), jnp.float32),
            pltpu.VMEM((HIDDEN, 128), jnp.float32),
            pltpu.SemaphoreType.DMA,
            pltpu.SemaphoreType.DMA,
            pltpu.SemaphoreType.DMA,
            pltpu.SemaphoreType.DMA,
        ],
        compiler_params=pltpu.CompilerParams(
            use_tc_tiling_on_sc=False, needs_layout_passes=False),
    )(x, token_table, pos_table)


def kernel(x, token_table, pos_table):
    out5 = _encode(x.astype(jnp.int32), token_table, pos_table)
    return out5.transpose(2, 4, 0, 1, 3).reshape(BATCH, SEQ_LEN, HIDDEN)
